# Initial kernel scaffold; baseline (speedup 1.0000x reference)
import functools

import jax
import jax.numpy as jnp
from jax.experimental import pallas as pl
from jax.experimental.pallas import tpu as pltpu

C = 64
NB = 8
SEG = 4096
QT = 512
KS = (8, 16, 32)
KMAX = 32
IMAX = jnp.int32(2147483647)


def _h_body(x_ref, w_ref, b_ref, g_ref, be_ref, h_ref):
    h = jnp.dot(x_ref[...], w_ref[...], preferred_element_type=jnp.float32)
    h = (h + b_ref[...]) * g_ref[...] + be_ref[...]
    h_ref[...] = jnp.maximum(h, 0.0)


def _sortable(d):
    i = jax.lax.bitcast_convert_type(d, jnp.int32)
    return jnp.where(i >= 0, i, jnp.bitwise_xor(jnp.bitwise_not(i),
                                                jnp.int32(-2147483648)))


def _knn_body(pt_ref, sq_ref, pq_ref, h_ref, out_ref):
    pt = pt_ref[0]
    sq = sq_ref[0]
    pq = pq_ref[0]
    hs = h_ref[0]
    sq_q = jnp.sum(pq * pq, axis=1, keepdims=True)
    d = sq_q + sq - 2.0 * jnp.dot(pq, pt,
                                  preferred_element_type=jnp.float32)
    keys = _sortable(d)
    wk = keys
    thr = {}
    for r in range(KMAX):
        m = jnp.min(wk, axis=1, keepdims=True)
        if (r + 1) in KS:
            thr[r + 1] = m
        wk = jnp.where(wk == m, IMAX, wk)
    for j, k in enumerate(KS):
        mask = (keys <= thr[k]).astype(jnp.float32)
        cnt = jnp.sum(mask, axis=1, keepdims=True)
        s = jnp.dot(mask, hs, preferred_element_type=jnp.float32)
        out_ref[0, :, j * C:(j + 1) * C] = s / cnt


def kernel(p, x, o, W, b, gamma, beta):
    n = p.shape[0]
    h = pl.pallas_call(
        _h_body,
        grid=(n // 2048,),
        in_specs=[
            pl.BlockSpec((2048, C), lambda i: (i, 0)),
            pl.BlockSpec((C, C), lambda i: (0, 0)),
            pl.BlockSpec((1, C), lambda i: (0, 0)),
            pl.BlockSpec((1, C), lambda i: (0, 0)),
            pl.BlockSpec((1, C), lambda i: (0, 0)),
        ],
        out_specs=pl.BlockSpec((2048, C), lambda i: (i, 0)),
        out_shape=jax.ShapeDtypeStruct((n, C), jnp.float32),
    )(x, W, b.reshape(1, C), gamma.reshape(1, C), beta.reshape(1, C))

    p3 = p.reshape(NB, SEG, 3)
    pt = jnp.transpose(p3, (0, 2, 1))
    sq = jnp.sum(p3 * p3, axis=2)[:, None, :]
    h3 = h.reshape(NB, SEG, C)

    knn = pl.pallas_call(
        _knn_body,
        grid=(NB, SEG // QT),
        in_specs=[
            pl.BlockSpec((1, 3, SEG), lambda s, q: (s, 0, 0)),
            pl.BlockSpec((1, 1, SEG), lambda s, q: (s, 0, 0)),
            pl.BlockSpec((1, QT, 3), lambda s, q: (s, q, 0)),
            pl.BlockSpec((1, SEG, C), lambda s, q: (s, 0, 0)),
        ],
        out_specs=pl.BlockSpec((1, QT, 3 * C), lambda s, q: (s, q, 0)),
        out_shape=jax.ShapeDtypeStruct((NB, SEG, 3 * C), jnp.float32),
        compiler_params=pltpu.CompilerParams(
            dimension_semantics=("arbitrary", "arbitrary"),
        ),
    )(pt, sq, p3, h3)

    out = jnp.concatenate([h, knn.reshape(n, 3 * C)], axis=1)
    return (p, out, o)


# speedup vs baseline: 17.6751x; 17.6751x over previous
import functools

import jax
import jax.numpy as jnp
from jax.experimental import pallas as pl
from jax.experimental.pallas import tpu as pltpu

C = 64
NB = 8
SEG = 4096
QT = 512
KS = (8, 16, 32)
KMAX = 32
IMAX = 2147483647


def _h_body(x_ref, w_ref, b_ref, g_ref, be_ref, h_ref):
    h = jnp.dot(x_ref[...], w_ref[...], preferred_element_type=jnp.float32)
    h = (h + b_ref[...]) * g_ref[...] + be_ref[...]
    h_ref[...] = jnp.maximum(h, 0.0)


def _sortable(d):
    i = jax.lax.bitcast_convert_type(d, jnp.int32)
    return jnp.where(i >= 0, i, jnp.bitwise_xor(jnp.bitwise_not(i),
                                                -2147483648))


def _knn_body(pt_ref, sq_ref, pq_ref, h_ref, out_ref):
    pt = pt_ref[0]
    sq = sq_ref[0]
    pq = pq_ref[0]
    hs = h_ref[0]
    sq_q = jnp.sum(pq * pq, axis=1, keepdims=True)
    d = sq_q + sq - 2.0 * jnp.dot(pq, pt,
                                  preferred_element_type=jnp.float32)
    keys = _sortable(d)
    wk = keys
    thr = {}
    for r in range(KMAX):
        m = jnp.min(wk, axis=1, keepdims=True)
        if (r + 1) in KS:
            thr[r + 1] = m
        wk = jnp.where(wk == m, IMAX, wk)
    for j, k in enumerate(KS):
        mask = (keys <= thr[k]).astype(jnp.float32)
        cnt = jnp.sum(mask, axis=1, keepdims=True)
        s = jnp.dot(mask, hs, preferred_element_type=jnp.float32)
        out_ref[0, :, j * C:(j + 1) * C] = s / cnt


def kernel(p, x, o, W, b, gamma, beta):
    n = p.shape[0]
    h = pl.pallas_call(
        _h_body,
        grid=(n // 2048,),
        in_specs=[
            pl.BlockSpec((2048, C), lambda i: (i, 0)),
            pl.BlockSpec((C, C), lambda i: (0, 0)),
            pl.BlockSpec((1, C), lambda i: (0, 0)),
            pl.BlockSpec((1, C), lambda i: (0, 0)),
            pl.BlockSpec((1, C), lambda i: (0, 0)),
        ],
        out_specs=pl.BlockSpec((2048, C), lambda i: (i, 0)),
        out_shape=jax.ShapeDtypeStruct((n, C), jnp.float32),
    )(x, W, b.reshape(1, C), gamma.reshape(1, C), beta.reshape(1, C))

    p3 = p.reshape(NB, SEG, 3)
    pt = jnp.transpose(p3, (0, 2, 1))
    sq = jnp.sum(p3 * p3, axis=2)[:, None, :]
    h3 = h.reshape(NB, SEG, C)

    knn = pl.pallas_call(
        _knn_body,
        grid=(NB, SEG // QT),
        in_specs=[
            pl.BlockSpec((1, 3, SEG), lambda s, q: (s, 0, 0)),
            pl.BlockSpec((1, 1, SEG), lambda s, q: (s, 0, 0)),
            pl.BlockSpec((1, QT, 3), lambda s, q: (s, q, 0)),
            pl.BlockSpec((1, SEG, C), lambda s, q: (s, 0, 0)),
        ],
        out_specs=pl.BlockSpec((1, QT, 3 * C), lambda s, q: (s, q, 0)),
        out_shape=jax.ShapeDtypeStruct((NB, SEG, 3 * C), jnp.float32),
        compiler_params=pltpu.CompilerParams(
            dimension_semantics=("arbitrary", "arbitrary"),
        ),
    )(pt, sq, p3, h3)

    out = jnp.concatenate([h, knn.reshape(n, 3 * C)], axis=1)
    return (p, out, o)
